# Initial kernel scaffold; baseline (speedup 1.0000x reference)
import jax
import jax.numpy as jnp
from jax.experimental import pallas as pl


def kernel(x, emb1_W, lin1_W, lin1_b, emb2_W, lin2_W, lin2_b, emb3_W, lin3_W, lin3_b):
    raise NotImplementedError("write your pallas kernel here")


# speedup vs baseline: 190.1404x; 190.1404x over previous
import functools

import jax
import jax.numpy as jnp
from jax import lax
from jax.experimental import pallas as pl
from jax.experimental.pallas import tpu as pltpu
from jax.experimental.pallas import tpu_sc as plsc

L = 16


def _sc_body(n_per_worker, chunk, num_cores,
             wtab_hbm, x_hbm, o1_hbm, o2_hbm, o3_hbm,
             w_v, t1_v, t2_v, t3_v, x_v, o1_v, o2_v, o3_v):
    wid = lax.axis_index("s") * num_cores + lax.axis_index("c")
    base = wid * n_per_worker

    pltpu.sync_copy(wtab_hbm, w_v)
    t1 = w_v[10]
    for d in range(5):
        t1 = t1 + w_v[d] * w_v[5 + d]
    t2 = w_v[21]
    for d in range(5):
        t2 = t2 + w_v[11 + d] * w_v[16 + d]
    z3 = w_v[42]
    for d in range(10):
        z3 = z3 + w_v[22 + d] * w_v[32 + d]
    ones = jnp.ones((L,), jnp.float32)
    t3 = ones / (ones + jnp.exp(-z3))
    t1_v[...] = t1
    t2_v[...] = t2
    t3_v[...] = t3

    G = 8
    iters = chunk // (L * G)

    def gather_group(i, _):
        start = i * (L * G)
        offs = [start + g * L for g in range(G)]
        idxs = [x_v[pl.ds(o, L)] for o in offs]
        r1 = [plsc.load_gather(t1_v, [idx]) for idx in idxs]
        r2 = [plsc.load_gather(t2_v, [idx]) for idx in idxs]
        r3 = [plsc.load_gather(t3_v, [idx]) for idx in idxs]
        for g in range(G):
            o1_v[pl.ds(offs[g], L)] = r1[g]
            o2_v[pl.ds(offs[g], L)] = r2[g]
            o3_v[pl.ds(offs[g], L)] = r3[g]
        return 0

    for s in range(n_per_worker // chunk):
        off = base + s * chunk
        pltpu.sync_copy(x_hbm.at[pl.ds(off, chunk)], x_v)
        lax.fori_loop(0, iters, gather_group, 0)
        pltpu.sync_copy(o1_v, o1_hbm.at[pl.ds(off, chunk)])
        pltpu.sync_copy(o2_v, o2_hbm.at[pl.ds(off, chunk)])
        pltpu.sync_copy(o3_v, o3_hbm.at[pl.ds(off, chunk)])


def kernel(x, emb1_W, lin1_W, lin1_b, emb2_W, lin2_W, lin2_b,
           emb3_W, lin3_W, lin3_b):
    B, Lseq = x.shape
    n = B * Lseq

    info = plsc.get_sparse_core_info()
    nw = info.num_cores * info.num_subcores
    n_per_worker = n // nw
    chunk = 12800
    assert n_per_worker % chunk == 0

    def colpack(emb_W, lin_W, lin_b):
        d = emb_W.shape[1]
        cols = jnp.zeros((d, L), jnp.float32).at[:, : emb_W.shape[0]].set(emb_W.T)
        lw = jnp.broadcast_to(lin_W[0][:, None], (d, L))
        bias = jnp.broadcast_to(lin_b[0], (1, L))
        return jnp.concatenate([cols, lw, bias], axis=0)

    wtab = jnp.concatenate(
        [colpack(emb1_W, lin1_W, lin1_b),
         colpack(emb2_W, lin2_W, lin2_b),
         colpack(emb3_W, lin3_W, lin3_b)], axis=0)

    mesh = plsc.VectorSubcoreMesh(core_axis_name="c", subcore_axis_name="s")
    f32 = jnp.float32
    out = pl.kernel(
        functools.partial(_sc_body, n_per_worker, chunk, info.num_cores),
        mesh=mesh,
        out_type=[jax.ShapeDtypeStruct((n,), f32)] * 3,
        scratch_types=[
            pltpu.VMEM((43, L), f32),
            pltpu.VMEM((L,), f32),
            pltpu.VMEM((L,), f32),
            pltpu.VMEM((L,), f32),
            pltpu.VMEM((chunk,), jnp.int32),
            pltpu.VMEM((chunk,), f32),
            pltpu.VMEM((chunk,), f32),
            pltpu.VMEM((chunk,), f32),
        ],
        compiler_params=pltpu.CompilerParams(needs_layout_passes=False),
    )(wtab, x.reshape(n).astype(jnp.int32))

    return tuple(o.reshape(B, Lseq, 1) for o in out)


# speedup vs baseline: 210.2924x; 1.1060x over previous
import functools

import jax
import jax.numpy as jnp
from jax import lax
from jax.experimental import pallas as pl
from jax.experimental.pallas import tpu as pltpu
from jax.experimental.pallas import tpu_sc as plsc

L = 16


def _sc_body(n_per_worker, chunk, num_cores,
             wtab_hbm, x_hbm, o1_hbm, o2_hbm, o3_hbm,
             w_v, t1_v, t2_v, t3_v,
             xa_v, xb_v, o1a_v, o2a_v, o3a_v, o1b_v, o2b_v, o3b_v,
             sia, sib, soa, sob):
    wid = lax.axis_index("s") * num_cores + lax.axis_index("c")
    base = wid * n_per_worker

    pltpu.sync_copy(wtab_hbm, w_v)
    t1 = w_v[10]
    for d in range(5):
        t1 = t1 + w_v[d] * w_v[5 + d]
    t2 = w_v[21]
    for d in range(5):
        t2 = t2 + w_v[11 + d] * w_v[16 + d]
    z3 = w_v[42]
    for d in range(10):
        z3 = z3 + w_v[22 + d] * w_v[32 + d]
    ones = jnp.ones((L,), jnp.float32)
    t3 = ones / (ones + jnp.exp(-z3))
    t1_v[...] = t1
    t2_v[...] = t2
    t3_v[...] = t3

    G = 8
    iters = chunk // (L * G)
    nsub = n_per_worker // chunk

    x_b = [xa_v, xb_v]
    o_b = [[o1a_v, o2a_v, o3a_v], [o1b_v, o2b_v, o3b_v]]
    o_hbm = [o1_hbm, o2_hbm, o3_hbm]
    sin = [sia, sib]
    sout = [soa, sob]

    def compute(x_v, o1_v, o2_v, o3_v):
        def gather_group(i, _):
            start = i * (L * G)
            offs = [start + g * L for g in range(G)]
            idxs = [x_v[pl.ds(o, L)] for o in offs]
            r1 = [plsc.load_gather(t1_v, [idx]) for idx in idxs]
            r2 = [plsc.load_gather(t2_v, [idx]) for idx in idxs]
            r3 = [plsc.load_gather(t3_v, [idx]) for idx in idxs]
            for g in range(G):
                o1_v[pl.ds(offs[g], L)] = r1[g]
                o2_v[pl.ds(offs[g], L)] = r2[g]
                o3_v[pl.ds(offs[g], L)] = r3[g]
            return 0
        lax.fori_loop(0, iters, gather_group, 0)

    pltpu.async_copy(x_hbm.at[pl.ds(base, chunk)], x_b[0], sin[0])

    for s in range(nsub):
        b = s % 2
        off = base + s * chunk
        if s + 1 < nsub:
            pltpu.async_copy(
                x_hbm.at[pl.ds(off + chunk, chunk)], x_b[1 - b], sin[1 - b])
        pltpu.make_async_copy(
            x_hbm.at[pl.ds(off, chunk)], x_b[b], sin[b]).wait()
        if s >= 2:
            prev = off - 2 * chunk
            for k in range(3):
                pltpu.make_async_copy(
                    o_b[b][k], o_hbm[k].at[pl.ds(prev, chunk)], sout[b]).wait()
        compute(x_b[b], *o_b[b])
        for k in range(3):
            pltpu.async_copy(o_b[b][k], o_hbm[k].at[pl.ds(off, chunk)], sout[b])

    for s in (nsub - 2, nsub - 1):
        b = s % 2
        off = base + s * chunk
        for k in range(3):
            pltpu.make_async_copy(
                o_b[b][k], o_hbm[k].at[pl.ds(off, chunk)], sout[b]).wait()


def kernel(x, emb1_W, lin1_W, lin1_b, emb2_W, lin2_W, lin2_b,
           emb3_W, lin3_W, lin3_b):
    B, Lseq = x.shape
    n = B * Lseq

    info = plsc.get_sparse_core_info()
    nw = info.num_cores * info.num_subcores
    n_per_worker = n // nw
    chunk = 12800
    assert n_per_worker % chunk == 0

    def colpack(emb_W, lin_W, lin_b):
        d = emb_W.shape[1]
        cols = jnp.zeros((d, L), jnp.float32).at[:, : emb_W.shape[0]].set(emb_W.T)
        lw = jnp.broadcast_to(lin_W[0][:, None], (d, L))
        bias = jnp.broadcast_to(lin_b[0], (1, L))
        return jnp.concatenate([cols, lw, bias], axis=0)

    wtab = jnp.concatenate(
        [colpack(emb1_W, lin1_W, lin1_b),
         colpack(emb2_W, lin2_W, lin2_b),
         colpack(emb3_W, lin3_W, lin3_b)], axis=0)

    mesh = plsc.VectorSubcoreMesh(core_axis_name="c", subcore_axis_name="s")
    f32 = jnp.float32
    out = pl.kernel(
        functools.partial(_sc_body, n_per_worker, chunk, info.num_cores),
        mesh=mesh,
        out_type=[jax.ShapeDtypeStruct((n,), f32)] * 3,
        scratch_types=[
            pltpu.VMEM((43, L), f32),
            pltpu.VMEM((L,), f32),
            pltpu.VMEM((L,), f32),
            pltpu.VMEM((L,), f32),
            pltpu.VMEM((chunk,), jnp.int32),
            pltpu.VMEM((chunk,), jnp.int32),
            pltpu.VMEM((chunk,), f32),
            pltpu.VMEM((chunk,), f32),
            pltpu.VMEM((chunk,), f32),
            pltpu.VMEM((chunk,), f32),
            pltpu.VMEM((chunk,), f32),
            pltpu.VMEM((chunk,), f32),
            pltpu.SemaphoreType.DMA,
            pltpu.SemaphoreType.DMA,
            pltpu.SemaphoreType.DMA,
            pltpu.SemaphoreType.DMA,
        ],
        compiler_params=pltpu.CompilerParams(needs_layout_passes=False),
    )(wtab, x.reshape(n).astype(jnp.int32))

    return tuple(o.reshape(B, Lseq, 1) for o in out)


# speedup vs baseline: 319.9549x; 1.5215x over previous
import functools

import jax
import jax.numpy as jnp
from jax import lax
from jax.experimental import pallas as pl
from jax.experimental.pallas import tpu as pltpu
from jax.experimental.pallas import tpu_sc as plsc

L = 16


def _sc_body(rows_per_worker, rblk, ncols, num_cores,
             wtab_hbm, x_hbm, o1_hbm, o2_hbm, o3_hbm,
             w_v, t1_v, t2_v, t3_v,
             xa_v, xb_v, o1a_v, o2a_v, o3a_v, o1b_v, o2b_v, o3b_v,
             sia, sib, soa, sob):
    wid = lax.axis_index("s") * num_cores + lax.axis_index("c")
    base = wid * rows_per_worker

    pltpu.sync_copy(wtab_hbm, w_v)
    t1 = w_v[10]
    for d in range(5):
        t1 = t1 + w_v[d] * w_v[5 + d]
    t2 = w_v[21]
    for d in range(5):
        t2 = t2 + w_v[11 + d] * w_v[16 + d]
    z3 = w_v[42]
    for d in range(10):
        z3 = z3 + w_v[22 + d] * w_v[32 + d]
    ones = jnp.ones((L,), jnp.float32)
    t3 = ones / (ones + jnp.exp(-z3))
    t1_v[...] = t1
    t2_v[...] = t2
    t3_v[...] = t3

    nsub = rows_per_worker // rblk

    cstarts = list(range(0, ncols - L + 1, L))
    if cstarts[-1] != ncols - L:
        cstarts.append(ncols - L)

    x_b = [xa_v, xb_v]
    o_b = [[o1a_v, o2a_v, o3a_v], [o1b_v, o2b_v, o3b_v]]
    o_hbm = [o1_hbm, o2_hbm, o3_hbm]
    sin = [sia, sib]
    sout = [soa, sob]

    def compute(x_v, o1_v, o2_v, o3_v):
        def row_body(r, _):
            idxs = [x_v[r, pl.ds(c, L)] for c in cstarts]
            r1 = [plsc.load_gather(t1_v, [idx]) for idx in idxs]
            r2 = [plsc.load_gather(t2_v, [idx]) for idx in idxs]
            r3 = [plsc.load_gather(t3_v, [idx]) for idx in idxs]
            for g, c in enumerate(cstarts):
                o1_v[r, pl.ds(c, L)] = r1[g]
                o2_v[r, pl.ds(c, L)] = r2[g]
                o3_v[r, pl.ds(c, L)] = r3[g]
            return 0
        lax.fori_loop(0, rblk, row_body, 0)

    pltpu.async_copy(x_hbm.at[pl.ds(base, rblk), :], x_b[0], sin[0])

    for s in range(nsub):
        b = s % 2
        row0 = base + s * rblk
        if s + 1 < nsub:
            pltpu.async_copy(
                x_hbm.at[pl.ds(row0 + rblk, rblk), :], x_b[1 - b], sin[1 - b])
        pltpu.make_async_copy(
            x_hbm.at[pl.ds(row0, rblk), :], x_b[b], sin[b]).wait()
        if s >= 2:
            prev = row0 - 2 * rblk
            for k in range(3):
                pltpu.make_async_copy(
                    o_b[b][k], o_hbm[k].at[pl.ds(prev, rblk), :], sout[b]).wait()
        compute(x_b[b], *o_b[b])
        for k in range(3):
            pltpu.async_copy(
                o_b[b][k], o_hbm[k].at[pl.ds(row0, rblk), :], sout[b])

    for s in (nsub - 2, nsub - 1):
        b = s % 2
        row0 = base + s * rblk
        for k in range(3):
            pltpu.make_async_copy(
                o_b[b][k], o_hbm[k].at[pl.ds(row0, rblk), :], sout[b]).wait()


def kernel(x, emb1_W, lin1_W, lin1_b, emb2_W, lin2_W, lin2_b,
           emb3_W, lin3_W, lin3_b):
    B, ncols = x.shape

    info = plsc.get_sparse_core_info()
    nw = info.num_cores * info.num_subcores
    rows_per_worker = B // nw
    rblk = 32
    assert rows_per_worker % rblk == 0

    def colpack(emb_W, lin_W, lin_b):
        d = emb_W.shape[1]
        cols = jnp.zeros((d, L), jnp.float32).at[:, : emb_W.shape[0]].set(emb_W.T)
        lw = jnp.broadcast_to(lin_W[0][:, None], (d, L))
        bias = jnp.broadcast_to(lin_b[0], (1, L))
        return jnp.concatenate([cols, lw, bias], axis=0)

    wtab = jnp.concatenate(
        [colpack(emb1_W, lin1_W, lin1_b),
         colpack(emb2_W, lin2_W, lin2_b),
         colpack(emb3_W, lin3_W, lin3_b)], axis=0)

    mesh = plsc.VectorSubcoreMesh(core_axis_name="c", subcore_axis_name="s")
    f32 = jnp.float32
    out = pl.kernel(
        functools.partial(_sc_body, rows_per_worker, rblk, ncols,
                          info.num_cores),
        mesh=mesh,
        out_type=[jax.ShapeDtypeStruct((B, ncols), f32)] * 3,
        scratch_types=[
            pltpu.VMEM((43, L), f32),
            pltpu.VMEM((L,), f32),
            pltpu.VMEM((L,), f32),
            pltpu.VMEM((L,), f32),
            pltpu.VMEM((rblk, ncols), jnp.int32),
            pltpu.VMEM((rblk, ncols), jnp.int32),
            pltpu.VMEM((rblk, ncols), f32),
            pltpu.VMEM((rblk, ncols), f32),
            pltpu.VMEM((rblk, ncols), f32),
            pltpu.VMEM((rblk, ncols), f32),
            pltpu.VMEM((rblk, ncols), f32),
            pltpu.VMEM((rblk, ncols), f32),
            pltpu.SemaphoreType.DMA,
            pltpu.SemaphoreType.DMA,
            pltpu.SemaphoreType.DMA,
            pltpu.SemaphoreType.DMA,
        ],
        compiler_params=pltpu.CompilerParams(
            needs_layout_passes=False, use_tc_tiling_on_sc=True),
    )(wtab, x)

    return tuple(o[:, :, None] for o in out)
